# baseline (speedup 1.0000x reference)
import functools

import jax
import jax.numpy as jnp
from jax import lax
from jax.experimental import pallas as pl
from jax.experimental.pallas import tpu as pltpu
from jax.experimental.pallas import tpu_sc as plsc

_B, _C, _P = 1024, 100000, 50
_PPAD = 64
_NW = 32
_RW = _B // _NW
_R = 16
_NX = 8
_CB = 12544
_CLAST = _C - (_NX - 1) * _CB


def _sc_gather_body(in_hbm, idx_hbm, flat_hbm, out_hbm, idx_v, row_v, val_v, sem):
    wid = lax.axis_index("s") * 2 + lax.axis_index("c")
    base = wid * _RW
    pltpu.sync_copy(idx_hbm.at[wid], idx_v)
    for r in range(_RW):
        pltpu.sync_copy(in_hbm.at[base + r], row_v)
        pltpu.sync_copy(row_v, flat_hbm.at[pl.ds((base + r) * _C, _C)])
    for r in range(_RW):
        pltpu.async_copy(flat_hbm.at[idx_v.at[r]], val_v.at[r], sem).wait()
    pltpu.sync_copy(val_v, out_hbm.at[wid])


@functools.cache
def _sc_gather():
    return pl.kernel(
        _sc_gather_body,
        mesh=plsc.VectorSubcoreMesh(core_axis_name="c", subcore_axis_name="s"),
        out_type=(
            jax.ShapeDtypeStruct((_B * _C,), jnp.float32),
            jax.ShapeDtypeStruct((_NW, _RW, _PPAD), jnp.float32),
        ),
        scratch_types=[
            pltpu.VMEM((_RW, _PPAD), jnp.int32),
            pltpu.VMEM((_C,), jnp.float32),
            pltpu.VMEM((_RW, _PPAD), jnp.float32),
            pltpu.SemaphoreType.DMA,
        ],
    )


def _dense_body(*refs):
    x_refs, o_ref = refs[:_NX], refs[_NX]

    def masked(j, x):
        if j < _NX - 1:
            return x
        tail = lax.broadcasted_iota(jnp.int32, (_R, _CB), 1) < _CLAST
        return jnp.where(tail, x, -jnp.inf)

    m = jnp.full((_R, 1), 0.0, dtype=jnp.float32)
    for j, xr in enumerate(x_refs):
        m = jnp.maximum(m, jnp.max(masked(j, xr[...]), axis=1, keepdims=True))
    s = jnp.exp(-m)
    for j, xr in enumerate(x_refs):
        s = s + jnp.sum(jnp.exp(masked(j, xr[...]) - m), axis=1, keepdims=True)
    o_ref[...] = m + jnp.log(s)


_dense_call = pl.pallas_call(
    _dense_body,
    grid=(_B // _R,),
    in_specs=[
        pl.BlockSpec((_R, _CB), functools.partial(lambda j, i: (i, j), j))
        for j in range(_NX)
    ],
    out_specs=pl.BlockSpec((_R, 1), lambda i: (i, 0)),
    out_shape=jax.ShapeDtypeStruct((_B, 1), jnp.float32),
)


def _comb_body(g_ref, al_ref, o_ref):
    g = g_ref[...]
    all_loss = al_ref[...]
    valid = lax.broadcasted_iota(jnp.int32, (_B, _PPAD), 1) < _P
    gm = jnp.where(valid, g, -jnp.inf)
    m_p = jnp.max(gm, axis=1, keepdims=True)
    s_p = jnp.sum(jnp.where(valid, jnp.exp(g - m_p), 0.0), axis=1, keepdims=True)
    lse_pos = m_p + jnp.log(s_p)

    z = jnp.where(valid, -g, -jnp.inf)
    m_n = jnp.maximum(jnp.max(z, axis=1, keepdims=True), 0.0)
    s_n = jnp.sum(jnp.where(valid, jnp.exp(-g - m_n), 0.0), axis=1, keepdims=True)
    pos_loss = m_n + jnp.log(s_n + jnp.exp(-m_n))

    aux = jnp.clip(1.0 - jnp.exp(lse_pos - all_loss), 1e-12, 1.0)
    o_ref[...] = pos_loss + all_loss + jnp.log(aux)


_comb_call = pl.pallas_call(
    _comb_body,
    out_shape=jax.ShapeDtypeStruct((_B, 1), jnp.float32),
)


def kernel(input, target):
    tgt = jnp.concatenate([target, target[:, : _PPAD - _P]], axis=1)
    flat_idx = tgt + (jnp.arange(_B, dtype=jnp.int32) * _C)[:, None]
    _, gathered = _sc_gather()(input, flat_idx.reshape(_NW, _RW, _PPAD))
    all_loss = _dense_call(*([input] * _NX))
    out = _comb_call(gathered.reshape(_B, _PPAD), all_loss)
    return out.reshape(_B)
